# baseline (speedup 1.0000x reference)
import functools

import jax
import jax.numpy as jnp
import numpy as np
from jax import lax
from jax.experimental import pallas as pl
from jax.experimental.pallas import tpu as pltpu
from jax.experimental.pallas import tpu_sc as plsc

_NODE_TYPES = ['hierarchy', 'protocol', 'impression', 'treatment']
_EDGE_TYPES = [
    ('protocol', 'is_children_of', 'hierarchy'),
    ('protocol', 'has', 'impression'),
    ('protocol', 'suggests', 'treatment'),
    ('hierarchy', 'is_parent_of', 'protocol'),
    ('impression', 'indicates', 'protocol'),
    ('treatment', 'is_suggested_by', 'protocol'),
]
_HEADS = 8
_DH = 16
_HID = 128
_BLK = 1000


def _et_name(et):
    return et[0] + '__' + et[1] + '__' + et[2]


def _mm_body(x_ref, w_ref, b_ref, o_ref, *, act):
    y = jnp.dot(x_ref[...], w_ref[...], preferred_element_type=jnp.float32)
    y = y + b_ref[...]
    if act == 'relu':
        y = jnp.maximum(y, 0.0)
    o_ref[...] = y


def _mm(x, w, b, act=None):
    n, d_in = x.shape
    d_out = w.shape[1]
    assert n % _BLK == 0, n
    return pl.pallas_call(
        functools.partial(_mm_body, act=act),
        grid=(n // _BLK,),
        in_specs=[
            pl.BlockSpec((_BLK, d_in), lambda i: (i, 0)),
            pl.BlockSpec((d_in, d_out), lambda i: (0, 0)),
            pl.BlockSpec((1, d_out), lambda i: (0, 0)),
        ],
        out_specs=pl.BlockSpec((_BLK, d_out), lambda i: (i, 0)),
        out_shape=jax.ShapeDtypeStruct((n, d_out), jnp.float32),
    )(x, w, b.reshape(1, d_out))


def _update_body(agg_ref, x_ref, w_ref, b_ref, a_ref, o_ref):
    x = agg_ref[...]
    g = 0.5 * x * (1.0 + jax.lax.erf(x * np.float32(1.0 / np.sqrt(2.0))))
    y = jnp.dot(g, w_ref[...], preferred_element_type=jnp.float32) + b_ref[...]
    a = a_ref[0, 0]
    o_ref[...] = a * y + (1.0 - a) * x_ref[...]


def _update(agg, x_old, w, b, a_scalar):
    n = agg.shape[0]
    assert n % _BLK == 0
    return pl.pallas_call(
        _update_body,
        grid=(n // _BLK,),
        in_specs=[
            pl.BlockSpec((_BLK, _HID), lambda i: (i, 0)),
            pl.BlockSpec((_BLK, _HID), lambda i: (i, 0)),
            pl.BlockSpec((_HID, _HID), lambda i: (0, 0)),
            pl.BlockSpec((1, _HID), lambda i: (0, 0)),
            pl.BlockSpec((1, 1), lambda i: (0, 0)),
        ],
        out_specs=pl.BlockSpec((_BLK, _HID), lambda i: (i, 0)),
        out_shape=jax.ShapeDtypeStruct((n, _HID), jnp.float32),
    )(agg, x_old, w, b.reshape(1, _HID), a_scalar.reshape(1, 1))


def _block_diag(rel):
    eye = jnp.eye(_HEADS, dtype=rel.dtype)
    big = jnp.einsum('hde,hg->hdge', rel, eye)
    return big.reshape(_HID, _HID)


_E = 400000
_EB = 128
_NBLK = _E // _EB
_NW = 32


_CMAX = 51200


@functools.lru_cache(maxsize=None)
def _make_edge_kernel(n_src, n_dst, lo, csz):
    chunked = not (lo == 0 and csz >= n_dst)
    slc = csz // 16
    mesh = plsc.VectorSubcoreMesh(core_axis_name="c", subcore_axis_name="s")

    @functools.partial(
        pl.kernel, mesh=mesh,
        compiler_params=pltpu.CompilerParams(use_tc_tiling_on_sc=False),
        out_type=jax.ShapeDtypeStruct((2, _HEADS, csz, 32), jnp.float32),
        scratch_types=[
            pltpu.VMEM((2, _EB), jnp.int32),
            pltpu.VMEM((2, _EB), jnp.int32),
            pltpu.VMEM((2, _EB), jnp.int32),
            pltpu.VMEM((2, _EB), jnp.int32),
            pltpu.VMEM((2, _EB, 32), jnp.float32),
            pltpu.VMEM((2, _EB, 16), jnp.float32),
            pltpu.VMEM((2, _EB, 32), jnp.float32),
            pltpu.VMEM_SHARED((csz + 16, 32), jnp.float32),
            pltpu.SemaphoreType.DMA,
            pltpu.SemaphoreType.DMA,
        ],
    )
    def ek(ktv_hbm, q_hbm, row_hbm, col_hbm, zeros_hbm, out_hbm,
           rowi, colh, colc, scol, ktvb, qb, msgb, acc, gsem, ssem):
        c = lax.axis_index("c")
        s = lax.axis_index("s")
        wid = s * 2 + c
        nblk = jnp.where(wid < _NBLK % _NW, _NBLK // _NW + 1, _NBLK // _NW)
        lanes = lax.iota(jnp.int32, 16)

        def prefetch(b, sl_, h):
            off = (wid + b * _NW) * _EB
            pltpu.sync_copy(row_hbm.at[pl.ds(off, _EB)], rowi.at[sl_])
            pltpu.sync_copy(col_hbm.at[pl.ds(off, _EB)], colc.at[sl_])
            for g2 in range(_EB // 16):
                gs = pl.ds(g2 * 16, 16)
                cv = colc[sl_, gs]
                rowi[sl_, gs] = rowi[sl_, gs] + h * n_src
                colh[sl_, gs] = cv + h * n_dst
                if chunked:
                    inr = (cv >= lo) & (cv < lo + csz)
                    colc[sl_, gs] = jnp.where(inr, cv - lo, csz)
                elif lo:
                    colc[sl_, gs] = cv - lo
            pltpu.async_copy(ktv_hbm.at[rowi.at[sl_]], ktvb.at[sl_], gsem)
            pltpu.async_copy(q_hbm.at[colh.at[sl_]], qb.at[sl_], gsem)

        def compute(sl_):
            def ebody(j8, _):
                for u in range(8):
                    j = j8 * 8 + u
                    k = ktvb[sl_, j, 0:16]
                    q = qb[sl_, j, :]
                    r = k * q
                    for sh in (1, 2, 4, 8):
                        r = r + jnp.take(r, jnp.bitwise_xor(lanes, sh))
                    e = jnp.exp(r)
                    msgb[sl_, j, 0:16] = ktvb[sl_, j, 16:32] * e
                    msgb[sl_, j, 16:32] = jnp.where(lanes == 0, e, 0.0)
                return 0

            lax.fori_loop(0, _EB // 8, ebody, 0)

        def wait_gathers(sl_):
            pltpu.make_async_copy(ktv_hbm.at[rowi.at[sl_]],
                                  ktvb.at[sl_], gsem).wait()
            pltpu.make_async_copy(q_hbm.at[colh.at[sl_]],
                                  qb.at[sl_], gsem).wait()

        def wait_scatter(sl_):
            pltpu.make_async_copy(msgb.at[sl_], acc.at[scol.at[sl_]],
                                  ssem).wait()

        def hbody(h, _):
            pltpu.sync_copy(zeros_hbm.at[pl.ds(s * slc, slc)],
                            acc.at[pl.ds(s * slc, slc)])
            prefetch(0, 0, h)
            plsc.subcore_barrier()

            def phase(b, cur, nxt):
                @pl.when(b < nblk)
                def _():
                    @pl.when(b + 1 < nblk)
                    def _():
                        prefetch(b + 1, nxt, h)
                    wait_gathers(cur)

                    @pl.when(b >= 2)
                    def _():
                        wait_scatter(cur)
                    compute(cur)
                    for g2 in range(_EB // 16):
                        gs = pl.ds(g2 * 16, 16)
                        scol[cur, gs] = colc[cur, gs]
                    pltpu.async_copy(msgb.at[cur], acc.at[scol.at[cur]],
                                     ssem, add=True)

            def bbody(i, _):
                phase(2 * i, 0, 1)
                phase(2 * i + 1, 1, 0)
                return 0

            lax.fori_loop(0, (_NBLK // _NW + 2) // 2, bbody, 0)
            wait_scatter(0)
            wait_scatter(1)
            plsc.subcore_barrier()
            pltpu.sync_copy(acc.at[pl.ds(s * slc, slc)],
                            out_hbm.at[c, h, pl.ds(s * slc, slc)])
            plsc.subcore_barrier()
            return 0

        lax.fori_loop(0, _HEADS, hbody, 0)

    return ek


def _head_major(a, n):
    return a.reshape(n, _HEADS, _DH).transpose(1, 0, 2)


def _dst_chunks(n_dst):
    n_pad = -(-n_dst // 2048) * 2048
    if n_pad <= _CMAX:
        return [(0, n_pad)]
    return [(0, _CMAX), (_CMAX, n_pad - _CMAX)]


def _edge_agg(kt, vt, q_hm, row, col, n_src, n_dst):
    ktv = jnp.concatenate(
        [_head_major(kt, n_src), _head_major(vt, n_src)],
        axis=-1).reshape(_HEADS * n_src, 2 * _DH)
    parts = []
    for lo, csz in _dst_chunks(n_dst):
        ek = _make_edge_kernel(n_src, n_dst, lo, csz)
        zeros = jnp.zeros((csz, 32), jnp.float32)
        out = ek(ktv, q_hm, row, col, zeros)
        tot = out[0] + out[1]
        rows = min(n_dst - lo, csz)
        num = tot[:, :rows, :16]
        den = tot[:, :rows, 16]
        parts.append(num / (den + 1e-30)[:, :, None])
    contrib = jnp.concatenate(parts, axis=1) if len(parts) > 1 else parts[0]
    return contrib.transpose(1, 0, 2).reshape(n_dst, _HID)


def kernel(x_hierarchy, x_protocol, x_impression, x_treatment, ei_protocol__is_children_of__hierarchy, ei_protocol__has__impression, ei_protocol__suggests__treatment, ei_hierarchy__is_parent_of__protocol, ei_impression__indicates__protocol, ei_treatment__is_suggested_by__protocol, lin_in_W_hierarchy, lin_in_b_hierarchy, lin_in_W_protocol, lin_in_b_protocol, lin_in_W_impression, lin_in_b_impression, lin_in_W_treatment, lin_in_b_treatment, L0_kW_hierarchy, L0_kb_hierarchy, L0_qW_hierarchy, L0_qb_hierarchy, L0_vW_hierarchy, L0_vb_hierarchy, L0_aW_hierarchy, L0_ab_hierarchy, L0_skip_hierarchy, L0_kW_protocol, L0_kb_protocol, L0_qW_protocol, L0_qb_protocol, L0_vW_protocol, L0_vb_protocol, L0_aW_protocol, L0_ab_protocol, L0_skip_protocol, L0_kW_impression, L0_kb_impression, L0_qW_impression, L0_qb_impression, L0_vW_impression, L0_vb_impression, L0_aW_impression, L0_ab_impression, L0_skip_impression, L0_kW_treatment, L0_kb_treatment, L0_qW_treatment, L0_qb_treatment, L0_vW_treatment, L0_vb_treatment, L0_aW_treatment, L0_ab_treatment, L0_skip_treatment, L0_arel_protocol__is_children_of__hierarchy, L0_mrel_protocol__is_children_of__hierarchy, L0_prel_protocol__is_children_of__hierarchy, L0_arel_protocol__has__impression, L0_mrel_protocol__has__impression, L0_prel_protocol__has__impression, L0_arel_protocol__suggests__treatment, L0_mrel_protocol__suggests__treatment, L0_prel_protocol__suggests__treatment, L0_arel_hierarchy__is_parent_of__protocol, L0_mrel_hierarchy__is_parent_of__protocol, L0_prel_hierarchy__is_parent_of__protocol, L0_arel_impression__indicates__protocol, L0_mrel_impression__indicates__protocol, L0_prel_impression__indicates__protocol, L0_arel_treatment__is_suggested_by__protocol, L0_mrel_treatment__is_suggested_by__protocol, L0_prel_treatment__is_suggested_by__protocol, L1_kW_hierarchy, L1_kb_hierarchy, L1_qW_hierarchy, L1_qb_hierarchy, L1_vW_hierarchy, L1_vb_hierarchy, L1_aW_hierarchy, L1_ab_hierarchy, L1_skip_hierarchy, L1_kW_protocol, L1_kb_protocol, L1_qW_protocol, L1_qb_protocol, L1_vW_protocol, L1_vb_protocol, L1_aW_protocol, L1_ab_protocol, L1_skip_protocol, L1_kW_impression, L1_kb_impression, L1_qW_impression, L1_qb_impression, L1_vW_impression, L1_vb_impression, L1_aW_impression, L1_ab_impression, L1_skip_impression, L1_kW_treatment, L1_kb_treatment, L1_qW_treatment, L1_qb_treatment, L1_vW_treatment, L1_vb_treatment, L1_aW_treatment, L1_ab_treatment, L1_skip_treatment, L1_arel_protocol__is_children_of__hierarchy, L1_mrel_protocol__is_children_of__hierarchy, L1_prel_protocol__is_children_of__hierarchy, L1_arel_protocol__has__impression, L1_mrel_protocol__has__impression, L1_prel_protocol__has__impression, L1_arel_protocol__suggests__treatment, L1_mrel_protocol__suggests__treatment, L1_prel_protocol__suggests__treatment, L1_arel_hierarchy__is_parent_of__protocol, L1_mrel_hierarchy__is_parent_of__protocol, L1_prel_hierarchy__is_parent_of__protocol, L1_arel_impression__indicates__protocol, L1_mrel_impression__indicates__protocol, L1_prel_impression__indicates__protocol, L1_arel_treatment__is_suggested_by__protocol, L1_mrel_treatment__is_suggested_by__protocol, L1_prel_treatment__is_suggested_by__protocol, lin_out_W, lin_out_b):
    p = dict(locals())

    xd = {nt: _mm(p['x_' + nt], p['lin_in_W_' + nt], p['lin_in_b_' + nt],
                  act='relu') for nt in _NODE_TYPES}
    eid = {_et_name(et): p['ei_' + _et_name(et)] for et in _EDGE_TYPES}

    for l in range(2):
        q_hm = {}
        for nt in _NODE_TYPES:
            n_nt = xd[nt].shape[0]
            q_hm[nt] = _head_major(
                _mm(xd[nt], p['L%d_qW_%s' % (l, nt)],
                    p['L%d_qb_%s' % (l, nt)]), n_nt).reshape(
                        _HEADS * n_nt, _DH)
        agg = {nt: jnp.zeros((xd[nt].shape[0], _HID), jnp.float32)
               for nt in _NODE_TYPES}
        for et in _EDGE_TYPES:
            src, _, dst = et
            en = _et_name(et)
            ei = eid[en]
            n_src = xd[src].shape[0]
            n_dst = xd[dst].shape[0]
            bd_a = _block_diag(p['L%d_arel_%s' % (l, en)])
            bd_m = _block_diag(p['L%d_mrel_%s' % (l, en)])
            prel = p['L%d_prel_%s' % (l, en)]
            scale = jnp.repeat(prel, _DH) * np.float32(1.0 / np.sqrt(_DH))
            kt_w = (p['L%d_kW_%s' % (l, src)] @ bd_a) * scale[None, :]
            kt_b = (p['L%d_kb_%s' % (l, src)] @ bd_a) * scale
            vt_w = p['L%d_vW_%s' % (l, src)] @ bd_m
            vt_b = p['L%d_vb_%s' % (l, src)] @ bd_m
            kt = _mm(xd[src], kt_w, kt_b)
            vt = _mm(xd[src], vt_w, vt_b)
            agg[dst] = agg[dst] + _edge_agg(kt, vt, q_hm[dst],
                                           ei[0], ei[1], n_src, n_dst)
        new_xd = {}
        for nt in _NODE_TYPES:
            new_xd[nt] = _update(agg[nt], xd[nt],
                                 p['L%d_aW_%s' % (l, nt)],
                                 p['L%d_ab_%s' % (l, nt)],
                                 jax.nn.sigmoid(p['L%d_skip_%s' % (l, nt)]))
        xd = new_xd

    return _mm(xd['protocol'], lin_out_W, lin_out_b)


# speedup vs baseline: 1.5588x; 1.5588x over previous
import functools

import jax
import jax.numpy as jnp
import numpy as np
from jax import lax
from jax.experimental import pallas as pl
from jax.experimental.pallas import tpu as pltpu
from jax.experimental.pallas import tpu_sc as plsc

_NODE_TYPES = ['hierarchy', 'protocol', 'impression', 'treatment']
_EDGE_TYPES = [
    ('protocol', 'is_children_of', 'hierarchy'),
    ('protocol', 'has', 'impression'),
    ('protocol', 'suggests', 'treatment'),
    ('hierarchy', 'is_parent_of', 'protocol'),
    ('impression', 'indicates', 'protocol'),
    ('treatment', 'is_suggested_by', 'protocol'),
]
_HEADS = 8
_DH = 16
_HID = 128
_BLK = 1000


def _et_name(et):
    return et[0] + '__' + et[1] + '__' + et[2]


def _mm_body(x_ref, w_ref, b_ref, o_ref, *, act):
    y = jnp.dot(x_ref[...], w_ref[...], preferred_element_type=jnp.float32)
    y = y + b_ref[...]
    if act == 'relu':
        y = jnp.maximum(y, 0.0)
    o_ref[...] = y


def _mm(x, w, b, act=None):
    n, d_in = x.shape
    d_out = w.shape[1]
    assert n % _BLK == 0, n
    return pl.pallas_call(
        functools.partial(_mm_body, act=act),
        grid=(n // _BLK,),
        in_specs=[
            pl.BlockSpec((_BLK, d_in), lambda i: (i, 0)),
            pl.BlockSpec((d_in, d_out), lambda i: (0, 0)),
            pl.BlockSpec((1, d_out), lambda i: (0, 0)),
        ],
        out_specs=pl.BlockSpec((_BLK, d_out), lambda i: (i, 0)),
        out_shape=jax.ShapeDtypeStruct((n, d_out), jnp.float32),
    )(x, w, b.reshape(1, d_out))


def _update_body(agg_ref, x_ref, w_ref, b_ref, a_ref, o_ref):
    x = agg_ref[...]
    g = 0.5 * x * (1.0 + jax.lax.erf(x * np.float32(1.0 / np.sqrt(2.0))))
    y = jnp.dot(g, w_ref[...], preferred_element_type=jnp.float32) + b_ref[...]
    a = a_ref[0, 0]
    o_ref[...] = a * y + (1.0 - a) * x_ref[...]


def _update(agg, x_old, w, b, a_scalar):
    n = agg.shape[0]
    assert n % _BLK == 0
    return pl.pallas_call(
        _update_body,
        grid=(n // _BLK,),
        in_specs=[
            pl.BlockSpec((_BLK, _HID), lambda i: (i, 0)),
            pl.BlockSpec((_BLK, _HID), lambda i: (i, 0)),
            pl.BlockSpec((_HID, _HID), lambda i: (0, 0)),
            pl.BlockSpec((1, _HID), lambda i: (0, 0)),
            pl.BlockSpec((1, 1), lambda i: (0, 0)),
        ],
        out_specs=pl.BlockSpec((_BLK, _HID), lambda i: (i, 0)),
        out_shape=jax.ShapeDtypeStruct((n, _HID), jnp.float32),
    )(agg, x_old, w, b.reshape(1, _HID), a_scalar.reshape(1, 1))


def _block_diag(rel):
    eye = jnp.eye(_HEADS, dtype=rel.dtype)
    big = jnp.einsum('hde,hg->hdge', rel, eye)
    return big.reshape(_HID, _HID)


_E = 400000
_EB = 128
_NBLK = _E // _EB
_NW = 32


@functools.lru_cache(maxsize=None)
def _make_edge_kernel(n_src, n_dst, csz):
    slc = csz // 16
    mesh = plsc.VectorSubcoreMesh(core_axis_name="c", subcore_axis_name="s")

    @functools.partial(
        pl.kernel, mesh=mesh,
        compiler_params=pltpu.CompilerParams(use_tc_tiling_on_sc=False),
        out_type=[
            jax.ShapeDtypeStruct((2, _HEADS, csz, 16), jnp.float32),
            jax.ShapeDtypeStruct((2, _HEADS, csz), jnp.float32),
        ],
        scratch_types=[
            pltpu.VMEM((2, _EB), jnp.int32),
            pltpu.VMEM((2, _EB), jnp.int32),
            pltpu.VMEM((2, _EB), jnp.int32),
            pltpu.VMEM((2, _EB), jnp.int32),
            pltpu.VMEM((2, _EB, 32), jnp.float32),
            pltpu.VMEM((2, _EB, 16), jnp.float32),
            pltpu.VMEM((2, _EB, 16), jnp.float32),
            pltpu.VMEM((2, _EB), jnp.float32),
            pltpu.VMEM_SHARED((csz + 16, 16), jnp.float32),
            pltpu.VMEM_SHARED((csz + 16,), jnp.float32),
            pltpu.SemaphoreType.DMA,
            pltpu.SemaphoreType.DMA,
        ],
    )
    def ek(ktv_hbm, q_hbm, row_hbm, col_hbm, zeros_hbm, zerod_hbm,
           outn_hbm, outd_hbm,
           rowi, colh, colc, scol, ktvb, qb, msgb, denb,
           accn, accd, gsem, ssem):
        c = lax.axis_index("c")
        s = lax.axis_index("s")
        wid = s * 2 + c
        nblk = jnp.where(wid < _NBLK % _NW, _NBLK // _NW + 1, _NBLK // _NW)
        lanes = lax.iota(jnp.int32, 16)

        def prefetch(b, sl_, h):
            off = (wid + b * _NW) * _EB
            pltpu.sync_copy(row_hbm.at[pl.ds(off, _EB)], rowi.at[sl_])
            pltpu.sync_copy(col_hbm.at[pl.ds(off, _EB)], colc.at[sl_])
            for g2 in range(_EB // 16):
                gs = pl.ds(g2 * 16, 16)
                rowi[sl_, gs] = rowi[sl_, gs] + h * n_src
                colh[sl_, gs] = colc[sl_, gs] + h * n_dst
            pltpu.async_copy(ktv_hbm.at[rowi.at[sl_]], ktvb.at[sl_], gsem)
            pltpu.async_copy(q_hbm.at[colh.at[sl_]], qb.at[sl_], gsem)

        def compute(sl_):
            def gbody(g, _):
                dv = jnp.zeros((16,), jnp.float32)
                for u in range(16):
                    j = g * 16 + u
                    k = ktvb[sl_, j, 0:16]
                    q = qb[sl_, j, :]
                    r = k * q
                    for sh in (1, 2, 4, 8):
                        r = r + jnp.take(r, jnp.bitwise_xor(lanes, sh))
                    e = jnp.exp(r)
                    msgb[sl_, j, :] = ktvb[sl_, j, 16:32] * e
                    dv = jnp.where(lanes == u, e, dv)
                denb[sl_, pl.ds(g * 16, 16)] = dv
                return 0

            lax.fori_loop(0, _EB // 16, gbody, 0)

        def wait_gathers(sl_):
            pltpu.make_async_copy(ktv_hbm.at[rowi.at[sl_]],
                                  ktvb.at[sl_], gsem).wait()
            pltpu.make_async_copy(q_hbm.at[colh.at[sl_]],
                                  qb.at[sl_], gsem).wait()

        def wait_scatter(sl_):
            pltpu.make_async_copy(msgb.at[sl_], accn.at[scol.at[sl_]],
                                  ssem).wait()
            pltpu.make_async_copy(denb.at[sl_], accd.at[scol.at[sl_]],
                                  ssem).wait()

        def hbody(h, _):
            pltpu.sync_copy(zeros_hbm.at[pl.ds(s * slc, slc)],
                            accn.at[pl.ds(s * slc, slc)])
            pltpu.sync_copy(zerod_hbm.at[pl.ds(s * slc, slc)],
                            accd.at[pl.ds(s * slc, slc)])
            prefetch(0, 0, h)
            plsc.subcore_barrier()

            def phase(b, cur, nxt):
                @pl.when(b < nblk)
                def _():
                    @pl.when(b + 1 < nblk)
                    def _():
                        prefetch(b + 1, nxt, h)
                    wait_gathers(cur)

                    @pl.when(b >= 2)
                    def _():
                        wait_scatter(cur)
                    compute(cur)
                    for g2 in range(_EB // 16):
                        gs = pl.ds(g2 * 16, 16)
                        scol[cur, gs] = colc[cur, gs]
                    pltpu.async_copy(msgb.at[cur], accn.at[scol.at[cur]],
                                     ssem, add=True)
                    pltpu.async_copy(denb.at[cur], accd.at[scol.at[cur]],
                                     ssem, add=True)

            def bbody(i, _):
                phase(2 * i, 0, 1)
                phase(2 * i + 1, 1, 0)
                return 0

            lax.fori_loop(0, (_NBLK // _NW + 2) // 2, bbody, 0)
            wait_scatter(0)
            wait_scatter(1)
            plsc.subcore_barrier()
            pltpu.sync_copy(accn.at[pl.ds(s * slc, slc)],
                            outn_hbm.at[c, h, pl.ds(s * slc, slc)])
            pltpu.sync_copy(accd.at[pl.ds(s * slc, slc)],
                            outd_hbm.at[c, h, pl.ds(s * slc, slc)])
            plsc.subcore_barrier()
            return 0

        lax.fori_loop(0, _HEADS, hbody, 0)

    return ek


def _head_major(a, n):
    return a.reshape(n, _HEADS, _DH).transpose(1, 0, 2)


def _edge_agg(kt, vt, q_hm, row, col, n_src, n_dst):
    ktv = jnp.concatenate(
        [_head_major(kt, n_src), _head_major(vt, n_src)],
        axis=-1).reshape(_HEADS * n_src, 2 * _DH)
    csz = -(-n_dst // 2048) * 2048
    ek = _make_edge_kernel(n_src, n_dst, csz)
    zeros = jnp.zeros((csz, 16), jnp.float32)
    zerod = jnp.zeros((csz,), jnp.float32)
    outn, outd = ek(ktv, q_hm, row, col, zeros, zerod)
    num = (outn[0] + outn[1])[:, :n_dst, :]
    den = (outd[0] + outd[1])[:, :n_dst]
    contrib = num / (den + 1e-30)[:, :, None]
    return contrib.transpose(1, 0, 2).reshape(n_dst, _HID)


def kernel(x_hierarchy, x_protocol, x_impression, x_treatment, ei_protocol__is_children_of__hierarchy, ei_protocol__has__impression, ei_protocol__suggests__treatment, ei_hierarchy__is_parent_of__protocol, ei_impression__indicates__protocol, ei_treatment__is_suggested_by__protocol, lin_in_W_hierarchy, lin_in_b_hierarchy, lin_in_W_protocol, lin_in_b_protocol, lin_in_W_impression, lin_in_b_impression, lin_in_W_treatment, lin_in_b_treatment, L0_kW_hierarchy, L0_kb_hierarchy, L0_qW_hierarchy, L0_qb_hierarchy, L0_vW_hierarchy, L0_vb_hierarchy, L0_aW_hierarchy, L0_ab_hierarchy, L0_skip_hierarchy, L0_kW_protocol, L0_kb_protocol, L0_qW_protocol, L0_qb_protocol, L0_vW_protocol, L0_vb_protocol, L0_aW_protocol, L0_ab_protocol, L0_skip_protocol, L0_kW_impression, L0_kb_impression, L0_qW_impression, L0_qb_impression, L0_vW_impression, L0_vb_impression, L0_aW_impression, L0_ab_impression, L0_skip_impression, L0_kW_treatment, L0_kb_treatment, L0_qW_treatment, L0_qb_treatment, L0_vW_treatment, L0_vb_treatment, L0_aW_treatment, L0_ab_treatment, L0_skip_treatment, L0_arel_protocol__is_children_of__hierarchy, L0_mrel_protocol__is_children_of__hierarchy, L0_prel_protocol__is_children_of__hierarchy, L0_arel_protocol__has__impression, L0_mrel_protocol__has__impression, L0_prel_protocol__has__impression, L0_arel_protocol__suggests__treatment, L0_mrel_protocol__suggests__treatment, L0_prel_protocol__suggests__treatment, L0_arel_hierarchy__is_parent_of__protocol, L0_mrel_hierarchy__is_parent_of__protocol, L0_prel_hierarchy__is_parent_of__protocol, L0_arel_impression__indicates__protocol, L0_mrel_impression__indicates__protocol, L0_prel_impression__indicates__protocol, L0_arel_treatment__is_suggested_by__protocol, L0_mrel_treatment__is_suggested_by__protocol, L0_prel_treatment__is_suggested_by__protocol, L1_kW_hierarchy, L1_kb_hierarchy, L1_qW_hierarchy, L1_qb_hierarchy, L1_vW_hierarchy, L1_vb_hierarchy, L1_aW_hierarchy, L1_ab_hierarchy, L1_skip_hierarchy, L1_kW_protocol, L1_kb_protocol, L1_qW_protocol, L1_qb_protocol, L1_vW_protocol, L1_vb_protocol, L1_aW_protocol, L1_ab_protocol, L1_skip_protocol, L1_kW_impression, L1_kb_impression, L1_qW_impression, L1_qb_impression, L1_vW_impression, L1_vb_impression, L1_aW_impression, L1_ab_impression, L1_skip_impression, L1_kW_treatment, L1_kb_treatment, L1_qW_treatment, L1_qb_treatment, L1_vW_treatment, L1_vb_treatment, L1_aW_treatment, L1_ab_treatment, L1_skip_treatment, L1_arel_protocol__is_children_of__hierarchy, L1_mrel_protocol__is_children_of__hierarchy, L1_prel_protocol__is_children_of__hierarchy, L1_arel_protocol__has__impression, L1_mrel_protocol__has__impression, L1_prel_protocol__has__impression, L1_arel_protocol__suggests__treatment, L1_mrel_protocol__suggests__treatment, L1_prel_protocol__suggests__treatment, L1_arel_hierarchy__is_parent_of__protocol, L1_mrel_hierarchy__is_parent_of__protocol, L1_prel_hierarchy__is_parent_of__protocol, L1_arel_impression__indicates__protocol, L1_mrel_impression__indicates__protocol, L1_prel_impression__indicates__protocol, L1_arel_treatment__is_suggested_by__protocol, L1_mrel_treatment__is_suggested_by__protocol, L1_prel_treatment__is_suggested_by__protocol, lin_out_W, lin_out_b):
    p = dict(locals())

    xd = {nt: _mm(p['x_' + nt], p['lin_in_W_' + nt], p['lin_in_b_' + nt],
                  act='relu') for nt in _NODE_TYPES}
    eid = {_et_name(et): p['ei_' + _et_name(et)] for et in _EDGE_TYPES}

    for l in range(2):
        q_hm = {}
        for nt in _NODE_TYPES:
            n_nt = xd[nt].shape[0]
            q_hm[nt] = _head_major(
                _mm(xd[nt], p['L%d_qW_%s' % (l, nt)],
                    p['L%d_qb_%s' % (l, nt)]), n_nt).reshape(
                        _HEADS * n_nt, _DH)
        agg = {nt: jnp.zeros((xd[nt].shape[0], _HID), jnp.float32)
               for nt in _NODE_TYPES}
        for et in _EDGE_TYPES:
            src, _, dst = et
            en = _et_name(et)
            ei = eid[en]
            n_src = xd[src].shape[0]
            n_dst = xd[dst].shape[0]
            bd_a = _block_diag(p['L%d_arel_%s' % (l, en)])
            bd_m = _block_diag(p['L%d_mrel_%s' % (l, en)])
            prel = p['L%d_prel_%s' % (l, en)]
            scale = jnp.repeat(prel, _DH) * np.float32(1.0 / np.sqrt(_DH))
            kt_w = (p['L%d_kW_%s' % (l, src)] @ bd_a) * scale[None, :]
            kt_b = (p['L%d_kb_%s' % (l, src)] @ bd_a) * scale
            vt_w = p['L%d_vW_%s' % (l, src)] @ bd_m
            vt_b = p['L%d_vb_%s' % (l, src)] @ bd_m
            kt = _mm(xd[src], kt_w, kt_b)
            vt = _mm(xd[src], vt_w, vt_b)
            agg[dst] = agg[dst] + _edge_agg(kt, vt, q_hm[dst],
                                           ei[0], ei[1], n_src, n_dst)
        new_xd = {}
        for nt in _NODE_TYPES:
            new_xd[nt] = _update(agg[nt], xd[nt],
                                 p['L%d_aW_%s' % (l, nt)],
                                 p['L%d_ab_%s' % (l, nt)],
                                 jax.nn.sigmoid(p['L%d_skip_%s' % (l, nt)]))
        xd = new_xd

    return _mm(xd['protocol'], lin_out_W, lin_out_b)


# speedup vs baseline: 1.7208x; 1.1040x over previous
import functools

import jax
import jax.numpy as jnp
import numpy as np
from jax import lax
from jax.experimental import pallas as pl
from jax.experimental.pallas import tpu as pltpu
from jax.experimental.pallas import tpu_sc as plsc

_NODE_TYPES = ['hierarchy', 'protocol', 'impression', 'treatment']
_EDGE_TYPES = [
    ('protocol', 'is_children_of', 'hierarchy'),
    ('protocol', 'has', 'impression'),
    ('protocol', 'suggests', 'treatment'),
    ('hierarchy', 'is_parent_of', 'protocol'),
    ('impression', 'indicates', 'protocol'),
    ('treatment', 'is_suggested_by', 'protocol'),
]
_HEADS = 8
_DH = 16
_HID = 128
_BLK = 1000


def _et_name(et):
    return et[0] + '__' + et[1] + '__' + et[2]


def _mm_body(x_ref, w_ref, b_ref, o_ref, *, act):
    y = jnp.dot(x_ref[...], w_ref[...], preferred_element_type=jnp.float32)
    y = y + b_ref[...]
    if act == 'relu':
        y = jnp.maximum(y, 0.0)
    o_ref[...] = y


def _mm(x, w, b, act=None):
    n, d_in = x.shape
    d_out = w.shape[1]
    assert n % _BLK == 0, n
    return pl.pallas_call(
        functools.partial(_mm_body, act=act),
        grid=(n // _BLK,),
        in_specs=[
            pl.BlockSpec((_BLK, d_in), lambda i: (i, 0)),
            pl.BlockSpec((d_in, d_out), lambda i: (0, 0)),
            pl.BlockSpec((1, d_out), lambda i: (0, 0)),
        ],
        out_specs=pl.BlockSpec((_BLK, d_out), lambda i: (i, 0)),
        out_shape=jax.ShapeDtypeStruct((n, d_out), jnp.float32),
    )(x, w, b.reshape(1, d_out))


def _update_body(agg_ref, x_ref, w_ref, b_ref, a_ref, o_ref):
    x = agg_ref[...]
    g = 0.5 * x * (1.0 + jax.lax.erf(x * np.float32(1.0 / np.sqrt(2.0))))
    y = jnp.dot(g, w_ref[...], preferred_element_type=jnp.float32) + b_ref[...]
    a = a_ref[0, 0]
    o_ref[...] = a * y + (1.0 - a) * x_ref[...]


def _update(agg, x_old, w, b, a_scalar):
    n = agg.shape[0]
    assert n % _BLK == 0
    return pl.pallas_call(
        _update_body,
        grid=(n // _BLK,),
        in_specs=[
            pl.BlockSpec((_BLK, _HID), lambda i: (i, 0)),
            pl.BlockSpec((_BLK, _HID), lambda i: (i, 0)),
            pl.BlockSpec((_HID, _HID), lambda i: (0, 0)),
            pl.BlockSpec((1, _HID), lambda i: (0, 0)),
            pl.BlockSpec((1, 1), lambda i: (0, 0)),
        ],
        out_specs=pl.BlockSpec((_BLK, _HID), lambda i: (i, 0)),
        out_shape=jax.ShapeDtypeStruct((n, _HID), jnp.float32),
    )(agg, x_old, w, b.reshape(1, _HID), a_scalar.reshape(1, 1))


def _block_diag(rel):
    eye = jnp.eye(_HEADS, dtype=rel.dtype)
    big = jnp.einsum('hde,hg->hdge', rel, eye)
    return big.reshape(_HID, _HID)


_E = 400000
_EB = 128
_NBLK = _E // _EB
_NW = 32
_SPAN = -(-_NBLK // _NW) * _EB
_EPAD = _NW * _SPAN


@functools.lru_cache(maxsize=None)
def _make_edge_kernel(n_src, n_dst, csz):
    slc = csz // 16
    mesh = plsc.VectorSubcoreMesh(core_axis_name="c", subcore_axis_name="s")

    @functools.partial(
        pl.kernel, mesh=mesh,
        compiler_params=pltpu.CompilerParams(use_tc_tiling_on_sc=False),
        out_type=[
            jax.ShapeDtypeStruct((2, _HEADS, csz, 16), jnp.float32),
            jax.ShapeDtypeStruct((2, _HEADS, csz), jnp.float32),
        ],
        scratch_types=[
            pltpu.VMEM((2, 2 * _EB), jnp.int32),
            pltpu.VMEM((2, 2 * _EB), jnp.int32),
            pltpu.VMEM((2, 2 * _EB), jnp.int32),
            pltpu.VMEM((2, _EB), jnp.int32),
            pltpu.VMEM((2, _EB, 32), jnp.float32),
            pltpu.VMEM((2, _EB, 16), jnp.float32),
            pltpu.VMEM((2, _EB, 16), jnp.float32),
            pltpu.VMEM((2, _EB), jnp.float32),
            pltpu.VMEM_SHARED((csz + 16, 16), jnp.float32),
            pltpu.VMEM_SHARED((csz + 16,), jnp.float32),
            pltpu.SemaphoreType.DMA,
            pltpu.SemaphoreType.DMA,
        ],
    )
    def ek(ktv_hbm, q_hbm, row_hbm, col_hbm, zeros_hbm, zerod_hbm,
           outn_hbm, outd_hbm,
           rowi, colh, colc, scol, ktvb, qb, msgb, denb,
           accn, accd, gsem, ssem):
        c = lax.axis_index("c")
        s = lax.axis_index("s")
        wid = s * 2 + c
        full = _E // _SPAN
        nblk = jnp.where(wid < full, _SPAN // _EB,
                         (_E - full * _SPAN) // _EB)
        lanes = lax.iota(jnp.int32, 16)

        def compute(sl_):
            def gbody(g, _):
                dv = jnp.zeros((16,), jnp.float32)
                for u in range(16):
                    j = g * 16 + u
                    k = ktvb[sl_, j, 0:16]
                    q = qb[sl_, j, :]
                    r = k * q
                    for sh in (1, 2, 4, 8):
                        r = r + jnp.take(r, jnp.bitwise_xor(lanes, sh))
                    e = jnp.exp(r)
                    msgb[sl_, j, :] = ktvb[sl_, j, 16:32] * e
                    dv = jnp.where(lanes == u, e, dv)
                denb[sl_, pl.ds(g * 16, 16)] = dv
                return 0

            lax.fori_loop(0, _EB // 16, gbody, 0)

        def wait_scatter(sl_):
            pltpu.make_async_copy(msgb.at[sl_], accn.at[scol.at[sl_]],
                                  ssem).wait()
            pltpu.make_async_copy(denb.at[sl_], accd.at[scol.at[sl_]],
                                  ssem).wait()

        def hbody(h, _):
            def load_idx(sup, isl):
                @pl.when(2 * sup < nblk)
                def _():
                    off = wid * _SPAN + sup * (2 * _EB)
                    pltpu.sync_copy(row_hbm.at[pl.ds(off, 2 * _EB)],
                                    rowi.at[isl])
                    pltpu.sync_copy(col_hbm.at[pl.ds(off, 2 * _EB)],
                                    colc.at[isl])
                    for g2 in range(2 * _EB // 16):
                        gs = pl.ds(g2 * 16, 16)
                        rowi[isl, gs] = rowi[isl, gs] + h * n_src
                        colh[isl, gs] = colc[isl, gs] + h * n_dst

            def fire_gathers(isl, ihalf, sl_):
                io = pl.ds(ihalf * _EB, _EB)
                pltpu.async_copy(ktv_hbm.at[rowi.at[isl, io]],
                                 ktvb.at[sl_], gsem)
                pltpu.async_copy(q_hbm.at[colh.at[isl, io]],
                                 qb.at[sl_], gsem)

            def wait_gathers(isl, ihalf, sl_):
                io = pl.ds(ihalf * _EB, _EB)
                pltpu.make_async_copy(ktv_hbm.at[rowi.at[isl, io]],
                                      ktvb.at[sl_], gsem).wait()
                pltpu.make_async_copy(q_hbm.at[colh.at[isl, io]],
                                      qb.at[sl_], gsem).wait()

            pltpu.sync_copy(zeros_hbm.at[pl.ds(s * slc, slc)],
                            accn.at[pl.ds(s * slc, slc)])
            pltpu.sync_copy(zerod_hbm.at[pl.ds(s * slc, slc)],
                            accd.at[pl.ds(s * slc, slc)])
            load_idx(0, 0)
            fire_gathers(0, 0, 0)
            plsc.subcore_barrier()

            def phase(b, isl, ihalf, nisl, nihalf, cur):
                @pl.when(b < nblk)
                def _():
                    @pl.when(b + 1 < nblk)
                    def _():
                        fire_gathers(nisl, nihalf, 1 - cur)
                    wait_gathers(isl, ihalf, cur)

                    @pl.when(b >= 2)
                    def _():
                        wait_scatter(cur)
                    compute(cur)
                    for g2 in range(_EB // 16):
                        gs = pl.ds(g2 * 16, 16)
                        scol[cur, gs] = colc[isl,
                                             pl.ds(ihalf * _EB + g2 * 16, 16)]
                    pltpu.async_copy(msgb.at[cur], accn.at[scol.at[cur]],
                                     ssem, add=True)
                    pltpu.async_copy(denb.at[cur], accd.at[scol.at[cur]],
                                     ssem, add=True)

            def bbody(j, _):
                b = 4 * j
                load_idx(2 * j + 1, 1)
                phase(b, 0, 0, 0, 1, 0)
                phase(b + 1, 0, 1, 1, 0, 1)
                load_idx(2 * j + 2, 0)
                phase(b + 2, 1, 0, 1, 1, 0)
                phase(b + 3, 1, 1, 0, 0, 1)
                return 0

            lax.fori_loop(0, (_SPAN // _EB + 3) // 4, bbody, 0)
            wait_scatter(0)
            wait_scatter(1)
            plsc.subcore_barrier()
            pltpu.sync_copy(accn.at[pl.ds(s * slc, slc)],
                            outn_hbm.at[c, h, pl.ds(s * slc, slc)])
            pltpu.sync_copy(accd.at[pl.ds(s * slc, slc)],
                            outd_hbm.at[c, h, pl.ds(s * slc, slc)])
            plsc.subcore_barrier()
            return 0

        lax.fori_loop(0, _HEADS, hbody, 0)

    return ek


def _head_major(a, n):
    return a.reshape(n, _HEADS, _DH).transpose(1, 0, 2)


def _edge_agg(kt, vt, q_hm, row, col, n_src, n_dst):
    ktv = jnp.concatenate(
        [_head_major(kt, n_src), _head_major(vt, n_src)],
        axis=-1).reshape(_HEADS * n_src, 2 * _DH)
    row = jnp.concatenate([row, jnp.zeros((_EPAD - _E,), jnp.int32)])
    col = jnp.concatenate([col, jnp.zeros((_EPAD - _E,), jnp.int32)])
    csz = -(-n_dst // 2048) * 2048
    ek = _make_edge_kernel(n_src, n_dst, csz)
    zeros = jnp.zeros((csz, 16), jnp.float32)
    zerod = jnp.zeros((csz,), jnp.float32)
    outn, outd = ek(ktv, q_hm, row, col, zeros, zerod)
    num = (outn[0] + outn[1])[:, :n_dst, :]
    den = (outd[0] + outd[1])[:, :n_dst]
    contrib = num / (den + 1e-30)[:, :, None]
    return contrib.transpose(1, 0, 2).reshape(n_dst, _HID)


def kernel(x_hierarchy, x_protocol, x_impression, x_treatment, ei_protocol__is_children_of__hierarchy, ei_protocol__has__impression, ei_protocol__suggests__treatment, ei_hierarchy__is_parent_of__protocol, ei_impression__indicates__protocol, ei_treatment__is_suggested_by__protocol, lin_in_W_hierarchy, lin_in_b_hierarchy, lin_in_W_protocol, lin_in_b_protocol, lin_in_W_impression, lin_in_b_impression, lin_in_W_treatment, lin_in_b_treatment, L0_kW_hierarchy, L0_kb_hierarchy, L0_qW_hierarchy, L0_qb_hierarchy, L0_vW_hierarchy, L0_vb_hierarchy, L0_aW_hierarchy, L0_ab_hierarchy, L0_skip_hierarchy, L0_kW_protocol, L0_kb_protocol, L0_qW_protocol, L0_qb_protocol, L0_vW_protocol, L0_vb_protocol, L0_aW_protocol, L0_ab_protocol, L0_skip_protocol, L0_kW_impression, L0_kb_impression, L0_qW_impression, L0_qb_impression, L0_vW_impression, L0_vb_impression, L0_aW_impression, L0_ab_impression, L0_skip_impression, L0_kW_treatment, L0_kb_treatment, L0_qW_treatment, L0_qb_treatment, L0_vW_treatment, L0_vb_treatment, L0_aW_treatment, L0_ab_treatment, L0_skip_treatment, L0_arel_protocol__is_children_of__hierarchy, L0_mrel_protocol__is_children_of__hierarchy, L0_prel_protocol__is_children_of__hierarchy, L0_arel_protocol__has__impression, L0_mrel_protocol__has__impression, L0_prel_protocol__has__impression, L0_arel_protocol__suggests__treatment, L0_mrel_protocol__suggests__treatment, L0_prel_protocol__suggests__treatment, L0_arel_hierarchy__is_parent_of__protocol, L0_mrel_hierarchy__is_parent_of__protocol, L0_prel_hierarchy__is_parent_of__protocol, L0_arel_impression__indicates__protocol, L0_mrel_impression__indicates__protocol, L0_prel_impression__indicates__protocol, L0_arel_treatment__is_suggested_by__protocol, L0_mrel_treatment__is_suggested_by__protocol, L0_prel_treatment__is_suggested_by__protocol, L1_kW_hierarchy, L1_kb_hierarchy, L1_qW_hierarchy, L1_qb_hierarchy, L1_vW_hierarchy, L1_vb_hierarchy, L1_aW_hierarchy, L1_ab_hierarchy, L1_skip_hierarchy, L1_kW_protocol, L1_kb_protocol, L1_qW_protocol, L1_qb_protocol, L1_vW_protocol, L1_vb_protocol, L1_aW_protocol, L1_ab_protocol, L1_skip_protocol, L1_kW_impression, L1_kb_impression, L1_qW_impression, L1_qb_impression, L1_vW_impression, L1_vb_impression, L1_aW_impression, L1_ab_impression, L1_skip_impression, L1_kW_treatment, L1_kb_treatment, L1_qW_treatment, L1_qb_treatment, L1_vW_treatment, L1_vb_treatment, L1_aW_treatment, L1_ab_treatment, L1_skip_treatment, L1_arel_protocol__is_children_of__hierarchy, L1_mrel_protocol__is_children_of__hierarchy, L1_prel_protocol__is_children_of__hierarchy, L1_arel_protocol__has__impression, L1_mrel_protocol__has__impression, L1_prel_protocol__has__impression, L1_arel_protocol__suggests__treatment, L1_mrel_protocol__suggests__treatment, L1_prel_protocol__suggests__treatment, L1_arel_hierarchy__is_parent_of__protocol, L1_mrel_hierarchy__is_parent_of__protocol, L1_prel_hierarchy__is_parent_of__protocol, L1_arel_impression__indicates__protocol, L1_mrel_impression__indicates__protocol, L1_prel_impression__indicates__protocol, L1_arel_treatment__is_suggested_by__protocol, L1_mrel_treatment__is_suggested_by__protocol, L1_prel_treatment__is_suggested_by__protocol, lin_out_W, lin_out_b):
    p = dict(locals())

    xd = {nt: _mm(p['x_' + nt], p['lin_in_W_' + nt], p['lin_in_b_' + nt],
                  act='relu') for nt in _NODE_TYPES}
    eid = {_et_name(et): p['ei_' + _et_name(et)] for et in _EDGE_TYPES}

    for l in range(2):
        q_hm = {}
        for nt in _NODE_TYPES:
            n_nt = xd[nt].shape[0]
            q_hm[nt] = _head_major(
                _mm(xd[nt], p['L%d_qW_%s' % (l, nt)],
                    p['L%d_qb_%s' % (l, nt)]), n_nt).reshape(
                        _HEADS * n_nt, _DH)
        agg = {nt: jnp.zeros((xd[nt].shape[0], _HID), jnp.float32)
               for nt in _NODE_TYPES}
        for et in _EDGE_TYPES:
            src, _, dst = et
            en = _et_name(et)
            ei = eid[en]
            n_src = xd[src].shape[0]
            n_dst = xd[dst].shape[0]
            bd_a = _block_diag(p['L%d_arel_%s' % (l, en)])
            bd_m = _block_diag(p['L%d_mrel_%s' % (l, en)])
            prel = p['L%d_prel_%s' % (l, en)]
            scale = jnp.repeat(prel, _DH) * np.float32(1.0 / np.sqrt(_DH))
            kt_w = (p['L%d_kW_%s' % (l, src)] @ bd_a) * scale[None, :]
            kt_b = (p['L%d_kb_%s' % (l, src)] @ bd_a) * scale
            vt_w = p['L%d_vW_%s' % (l, src)] @ bd_m
            vt_b = p['L%d_vb_%s' % (l, src)] @ bd_m
            kt = _mm(xd[src], kt_w, kt_b)
            vt = _mm(xd[src], vt_w, vt_b)
            agg[dst] = agg[dst] + _edge_agg(kt, vt, q_hm[dst],
                                           ei[0], ei[1], n_src, n_dst)
        new_xd = {}
        for nt in _NODE_TYPES:
            new_xd[nt] = _update(agg[nt], xd[nt],
                                 p['L%d_aW_%s' % (l, nt)],
                                 p['L%d_ab_%s' % (l, nt)],
                                 jax.nn.sigmoid(p['L%d_skip_%s' % (l, nt)]))
        xd = new_xd

    return _mm(xd['protocol'], lin_out_W, lin_out_b)
